# Initial kernel scaffold; baseline (speedup 1.0000x reference)
import functools

import jax
import jax.numpy as jnp
from jax import lax
from jax.experimental import pallas as pl
from jax.experimental.pallas import tpu as pltpu
from jax.experimental.pallas import tpu_sc as plsc

N = 10000
E = 320000
D = 128
H = 256
DA = 144
NC = 2
NS = 16
CHUNK = 80
ROWS_PER_SUB = N // NS
ZCH = 125
RB = 1000


def _sc_agg1(x_aug, src, dst):
    eps = E // (NC * NS)
    nch = eps // CHUNK
    mesh = plsc.VectorSubcoreMesh(core_axis_name="c", subcore_axis_name="s")

    @functools.partial(
        pl.kernel,
        out_type=jax.ShapeDtypeStruct((NC, N, DA), jnp.float32),
        mesh=mesh,
        scratch_types=[
            pltpu.VMEM((CHUNK,), jnp.int32),
            pltpu.VMEM((CHUNK,), jnp.int32),
            pltpu.VMEM((CHUNK, DA), jnp.float32),
            pltpu.VMEM((ZCH, DA), jnp.float32),
            pltpu.VMEM_SHARED((N, DA), jnp.float32),
            pltpu.SemaphoreType.DMA,
        ],
    )
    def k(x_hbm, src_hbm, dst_hbm, out_hbm, sidx, didx, rows, zbuf, acc, sem):
        c = lax.axis_index("c")
        s = lax.axis_index("s")

        @pl.loop(0, ZCH)
        def _(r):
            @pl.loop(0, DA, step=16)
            def _(j):
                zbuf[r, pl.ds(j, 16)] = jnp.zeros((16,), jnp.float32)

        @pl.loop(0, ROWS_PER_SUB // ZCH)
        def _(j):
            pltpu.sync_copy(zbuf,
                            acc.at[pl.ds(s * ROWS_PER_SUB + j * ZCH, ZCH)])

        plsc.subcore_barrier()

        base = (c * NS + s) * eps

        @pl.loop(0, nch)
        def _(i):
            e0 = base + i * CHUNK
            pltpu.sync_copy(src_hbm.at[pl.ds(e0, CHUNK)], sidx)
            pltpu.sync_copy(dst_hbm.at[pl.ds(e0, CHUNK)], didx)
            pltpu.async_copy(x_hbm.at[sidx], rows, sem).wait()
            pltpu.sync_copy(rows, acc.at[didx], add=True)

        plsc.subcore_barrier()
        r0 = s * ROWS_PER_SUB
        pltpu.sync_copy(acc.at[pl.ds(r0, ROWS_PER_SUB)],
                        out_hbm.at[c, pl.ds(r0, ROWS_PER_SUB)])

    return k(x_aug, src, dst)


def _sc_agg2(h0, h1, src, dst):
    eps = E // NS
    nch = eps // CHUNK
    mesh = plsc.VectorSubcoreMesh(core_axis_name="c", subcore_axis_name="s")

    @functools.partial(
        pl.kernel,
        out_type=jax.ShapeDtypeStruct((NC, N, D), jnp.float32),
        mesh=mesh,
        scratch_types=[
            pltpu.VMEM((CHUNK,), jnp.int32),
            pltpu.VMEM((CHUNK,), jnp.int32),
            pltpu.VMEM((CHUNK, D), jnp.float32),
            pltpu.VMEM((ZCH, D), jnp.float32),
            pltpu.VMEM_SHARED((N, D), jnp.float32),
            pltpu.SemaphoreType.DMA,
        ],
    )
    def k(h0_hbm, h1_hbm, src_hbm, dst_hbm, out_hbm,
          sidx, didx, rows, zbuf, acc, sem):
        c = lax.axis_index("c")
        s = lax.axis_index("s")

        @pl.loop(0, ZCH)
        def _(r):
            @pl.loop(0, D, step=16)
            def _(j):
                zbuf[r, pl.ds(j, 16)] = jnp.zeros((16,), jnp.float32)

        @pl.loop(0, ROWS_PER_SUB // ZCH)
        def _(j):
            pltpu.sync_copy(zbuf,
                            acc.at[pl.ds(s * ROWS_PER_SUB + j * ZCH, ZCH)])

        plsc.subcore_barrier()

        base = s * eps

        @pl.loop(0, nch)
        def _(i):
            e0 = base + i * CHUNK
            pltpu.sync_copy(src_hbm.at[pl.ds(e0, CHUNK)], sidx)
            pltpu.sync_copy(dst_hbm.at[pl.ds(e0, CHUNK)], didx)

            @pl.when(c == 0)
            def _():
                pltpu.async_copy(h0_hbm.at[sidx], rows, sem).wait()

            @pl.when(c == 1)
            def _():
                pltpu.async_copy(h1_hbm.at[sidx], rows, sem).wait()

            pltpu.sync_copy(rows, acc.at[didx], add=True)

        plsc.subcore_barrier()
        r0 = s * ROWS_PER_SUB
        pltpu.sync_copy(acc.at[pl.ds(r0, ROWS_PER_SUB)],
                        out_hbm.at[c, pl.ds(r0, ROWS_PER_SUB)])

    return k(h0, h1, src, dst)


def _tc_layer1(agg1, x, W1l, b1, W1r):
    def body(a_ref, x_ref, wl_ref, b_ref, wr_ref, h0_ref, h1_ref):
        a0 = a_ref[0]
        a1 = a_ref[1]
        ssum = a0[:, :D] + a1[:, :D]
        cnt = a0[:, D] + a1[:, D]
        mean = ssum / jnp.clip(cnt, 1.0)[:, None]
        h = jnp.dot(mean, wl_ref[...].T, preferred_element_type=jnp.float32)
        h = h + jnp.dot(x_ref[...], wr_ref[...].T,
                        preferred_element_type=jnp.float32)
        h = jnp.maximum(h + b_ref[...], 0.0)
        h0_ref[...] = h[:, :D]
        h1_ref[...] = h[:, D:]

    return pl.pallas_call(
        body,
        grid=(N // RB,),
        in_specs=[
            pl.BlockSpec((NC, RB, DA), lambda i: (0, i, 0)),
            pl.BlockSpec((RB, D), lambda i: (i, 0)),
            pl.BlockSpec((H, D), lambda i: (0, 0)),
            pl.BlockSpec((1, H), lambda i: (0, 0)),
            pl.BlockSpec((H, D), lambda i: (0, 0)),
        ],
        out_specs=[
            pl.BlockSpec((RB, D), lambda i: (i, 0)),
            pl.BlockSpec((RB, D), lambda i: (i, 0)),
        ],
        out_shape=[jax.ShapeDtypeStruct((N, D), jnp.float32),
                   jax.ShapeDtypeStruct((N, D), jnp.float32)],
    )(agg1, x, W1l, b1.reshape(1, H), W1r)


def _tc_layer2(agg2, agg1, h0, h1, W2l, b2, W2r):
    def body(g_ref, a_ref, h0_ref, h1_ref, wl_ref, b_ref, wr_ref, o_ref):
        cnt = a_ref[0][:, D] + a_ref[1][:, D]
        inv = 1.0 / jnp.clip(cnt, 1.0)
        m0 = g_ref[0] * inv[:, None]
        m1 = g_ref[1] * inv[:, None]
        wl = wl_ref[...]
        wr = wr_ref[...]
        o = jnp.dot(m0, wl[:, :D].T, preferred_element_type=jnp.float32)
        o = o + jnp.dot(m1, wl[:, D:].T, preferred_element_type=jnp.float32)
        o = o + jnp.dot(h0_ref[...], wr[:, :D].T,
                        preferred_element_type=jnp.float32)
        o = o + jnp.dot(h1_ref[...], wr[:, D:].T,
                        preferred_element_type=jnp.float32)
        o_ref[...] = o + b_ref[...]

    return pl.pallas_call(
        body,
        grid=(N // RB,),
        in_specs=[
            pl.BlockSpec((NC, RB, D), lambda i: (0, i, 0)),
            pl.BlockSpec((NC, RB, DA), lambda i: (0, i, 0)),
            pl.BlockSpec((RB, D), lambda i: (i, 0)),
            pl.BlockSpec((RB, D), lambda i: (i, 0)),
            pl.BlockSpec((H, H), lambda i: (0, 0)),
            pl.BlockSpec((1, H), lambda i: (0, 0)),
            pl.BlockSpec((H, H), lambda i: (0, 0)),
        ],
        out_specs=pl.BlockSpec((RB, H), lambda i: (i, 0)),
        out_shape=jax.ShapeDtypeStruct((N, H), jnp.float32),
    )(agg2, agg1, h0, h1, W2l, b2.reshape(1, H), W2r)


def kernel(x, edge_index, W1l, b1, W1r, W2l, b2, W2r):
    ei = edge_index.astype(jnp.int32)
    src = ei[0]
    dst = ei[1]
    x_aug = jnp.concatenate(
        [x, jnp.ones((N, 1), jnp.float32),
         jnp.zeros((N, DA - D - 1), jnp.float32)], axis=1)
    agg1 = _sc_agg1(x_aug, src, dst)
    h0, h1 = _tc_layer1(agg1, x, W1l, b1, W1r)
    agg2 = _sc_agg2(h0, h1, src, dst)
    return _tc_layer2(agg2, agg1, h0, h1, W2l, b2, W2r)


# speedup vs baseline: 4.6509x; 4.6509x over previous
import functools

import jax
import jax.numpy as jnp
from jax import lax
from jax.experimental import pallas as pl
from jax.experimental.pallas import tpu as pltpu
from jax.experimental.pallas import tpu_sc as plsc

N = 10000
E = 320000
D = 128
H = 256
NC = 2
NS = 16
CHUNK = 80
NPAD = 10240
ROWS_PER_SUB = NPAD // NS
ZCH = 128
RB = 1280


def _sc_agg1(x, src, dst):
    eps = E // (NC * NS)
    nch = eps // CHUNK
    mesh = plsc.VectorSubcoreMesh(core_axis_name="c", subcore_axis_name="s")

    @functools.partial(
        pl.kernel,
        out_type=[jax.ShapeDtypeStruct((NC, NPAD, D), jnp.float32),
                  jax.ShapeDtypeStruct((NC, NPAD), jnp.float32)],
        mesh=mesh,
        scratch_types=[
            pltpu.VMEM((CHUNK,), jnp.int32),
            pltpu.VMEM((CHUNK,), jnp.int32),
            pltpu.VMEM((CHUNK, D), jnp.float32),
            pltpu.VMEM((CHUNK,), jnp.float32),
            pltpu.VMEM((ZCH, D), jnp.float32),
            pltpu.VMEM_SHARED((NPAD, D), jnp.float32),
            pltpu.VMEM_SHARED((NPAD,), jnp.float32),
            pltpu.SemaphoreType.DMA,
        ],
    )
    def k(x_hbm, src_hbm, dst_hbm, osum_hbm, ocnt_hbm,
          sidx, didx, rows, ones, zbuf, acc, acc_cnt, sem):
        c = lax.axis_index("c")
        s = lax.axis_index("s")

        @pl.loop(0, CHUNK, step=16)
        def _(j):
            ones[pl.ds(j, 16)] = jnp.ones((16,), jnp.float32)

        @pl.loop(0, ZCH)
        def _(r):
            @pl.loop(0, D, step=16)
            def _(j):
                zbuf[r, pl.ds(j, 16)] = jnp.zeros((16,), jnp.float32)

        @pl.loop(0, ROWS_PER_SUB // ZCH)
        def _(j):
            pltpu.sync_copy(zbuf,
                            acc.at[pl.ds(s * ROWS_PER_SUB + j * ZCH, ZCH)])

        pltpu.sync_copy(zbuf.at[0, pl.ds(0, ZCH)],
                        acc_cnt.at[pl.ds(s * ROWS_PER_SUB, ZCH)])
        pltpu.sync_copy(zbuf.at[1, pl.ds(0, ZCH)],
                        acc_cnt.at[pl.ds(s * ROWS_PER_SUB + ZCH, ZCH)])
        pltpu.sync_copy(zbuf.at[2, pl.ds(0, ZCH)],
                        acc_cnt.at[pl.ds(s * ROWS_PER_SUB + 2 * ZCH, ZCH)])
        pltpu.sync_copy(zbuf.at[3, pl.ds(0, ZCH)],
                        acc_cnt.at[pl.ds(s * ROWS_PER_SUB + 3 * ZCH, ZCH)])
        pltpu.sync_copy(zbuf.at[4, pl.ds(0, ZCH)],
                        acc_cnt.at[pl.ds(s * ROWS_PER_SUB + 4 * ZCH, ZCH)])

        plsc.subcore_barrier()

        base = (c * NS + s) * eps

        @pl.loop(0, nch)
        def _(i):
            e0 = base + i * CHUNK
            pltpu.sync_copy(src_hbm.at[pl.ds(e0, CHUNK)], sidx)
            pltpu.sync_copy(dst_hbm.at[pl.ds(e0, CHUNK)], didx)
            pltpu.async_copy(x_hbm.at[sidx], rows, sem).wait()
            pltpu.sync_copy(rows, acc.at[didx], add=True)
            pltpu.sync_copy(ones, acc_cnt.at[didx], add=True)

        plsc.subcore_barrier()
        r0 = s * ROWS_PER_SUB
        pltpu.sync_copy(acc.at[pl.ds(r0, ROWS_PER_SUB)],
                        osum_hbm.at[c, pl.ds(r0, ROWS_PER_SUB)])
        pltpu.sync_copy(acc_cnt.at[pl.ds(r0, ROWS_PER_SUB)],
                        ocnt_hbm.at[c, pl.ds(r0, ROWS_PER_SUB)])

    return k(x, src, dst)


def _sc_agg2(h0, h1, src, dst):
    eps = E // NS
    nch = eps // CHUNK
    mesh = plsc.VectorSubcoreMesh(core_axis_name="c", subcore_axis_name="s")

    @functools.partial(
        pl.kernel,
        out_type=jax.ShapeDtypeStruct((NC, NPAD, D), jnp.float32),
        mesh=mesh,
        scratch_types=[
            pltpu.VMEM((CHUNK,), jnp.int32),
            pltpu.VMEM((CHUNK,), jnp.int32),
            pltpu.VMEM((CHUNK, D), jnp.float32),
            pltpu.VMEM((ZCH, D), jnp.float32),
            pltpu.VMEM_SHARED((NPAD, D), jnp.float32),
            pltpu.SemaphoreType.DMA,
        ],
    )
    def k(h0_hbm, h1_hbm, src_hbm, dst_hbm, out_hbm,
          sidx, didx, rows, zbuf, acc, sem):
        c = lax.axis_index("c")
        s = lax.axis_index("s")

        @pl.loop(0, ZCH)
        def _(r):
            @pl.loop(0, D, step=16)
            def _(j):
                zbuf[r, pl.ds(j, 16)] = jnp.zeros((16,), jnp.float32)

        @pl.loop(0, ROWS_PER_SUB // ZCH)
        def _(j):
            pltpu.sync_copy(zbuf,
                            acc.at[pl.ds(s * ROWS_PER_SUB + j * ZCH, ZCH)])

        plsc.subcore_barrier()

        base = s * eps

        @pl.loop(0, nch)
        def _(i):
            e0 = base + i * CHUNK
            pltpu.sync_copy(src_hbm.at[pl.ds(e0, CHUNK)], sidx)
            pltpu.sync_copy(dst_hbm.at[pl.ds(e0, CHUNK)], didx)

            @pl.when(c == 0)
            def _():
                pltpu.async_copy(h0_hbm.at[sidx], rows, sem).wait()

            @pl.when(c == 1)
            def _():
                pltpu.async_copy(h1_hbm.at[sidx], rows, sem).wait()

            pltpu.sync_copy(rows, acc.at[didx], add=True)

        plsc.subcore_barrier()
        r0 = s * ROWS_PER_SUB
        pltpu.sync_copy(acc.at[pl.ds(r0, ROWS_PER_SUB)],
                        out_hbm.at[c, pl.ds(r0, ROWS_PER_SUB)])

    return k(h0, h1, src, dst)


def _tc_layer1(sum1, cnt, x, W1l, b1, W1r):
    def body(a_ref, c_ref, x_ref, wl_ref, b_ref, wr_ref, h0_ref, h1_ref):
        ssum = a_ref[0] + a_ref[1]
        deg = c_ref[0] + c_ref[1]
        mean = ssum / jnp.clip(deg, 1.0)[:, None]
        h = jnp.dot(mean, wl_ref[...].T, preferred_element_type=jnp.float32)
        h = h + jnp.dot(x_ref[...], wr_ref[...].T,
                        preferred_element_type=jnp.float32)
        h = jnp.maximum(h + b_ref[...], 0.0)
        h0_ref[...] = h[:, :D]
        h1_ref[...] = h[:, D:]

    return pl.pallas_call(
        body,
        grid=(pl.cdiv(N, RB),),
        in_specs=[
            pl.BlockSpec((NC, RB, D), lambda i: (0, i, 0)),
            pl.BlockSpec((NC, RB), lambda i: (0, i)),
            pl.BlockSpec((RB, D), lambda i: (i, 0)),
            pl.BlockSpec((H, D), lambda i: (0, 0)),
            pl.BlockSpec((1, H), lambda i: (0, 0)),
            pl.BlockSpec((H, D), lambda i: (0, 0)),
        ],
        out_specs=[
            pl.BlockSpec((RB, D), lambda i: (i, 0)),
            pl.BlockSpec((RB, D), lambda i: (i, 0)),
        ],
        out_shape=[jax.ShapeDtypeStruct((N, D), jnp.float32),
                   jax.ShapeDtypeStruct((N, D), jnp.float32)],
    )(sum1, cnt, x, W1l, b1.reshape(1, H), W1r)


def _tc_layer2(agg2, cnt, h0, h1, W2l, b2, W2r):
    def body(g_ref, c_ref, h0_ref, h1_ref, wl_ref, b_ref, wr_ref, o_ref):
        deg = c_ref[0] + c_ref[1]
        inv = 1.0 / jnp.clip(deg, 1.0)
        m0 = g_ref[0] * inv[:, None]
        m1 = g_ref[1] * inv[:, None]
        wl = wl_ref[...]
        wr = wr_ref[...]
        o = jnp.dot(m0, wl[:, :D].T, preferred_element_type=jnp.float32)
        o = o + jnp.dot(m1, wl[:, D:].T, preferred_element_type=jnp.float32)
        o = o + jnp.dot(h0_ref[...], wr[:, :D].T,
                        preferred_element_type=jnp.float32)
        o = o + jnp.dot(h1_ref[...], wr[:, D:].T,
                        preferred_element_type=jnp.float32)
        o_ref[...] = o + b_ref[...]

    return pl.pallas_call(
        body,
        grid=(pl.cdiv(N, RB),),
        in_specs=[
            pl.BlockSpec((NC, RB, D), lambda i: (0, i, 0)),
            pl.BlockSpec((NC, RB), lambda i: (0, i)),
            pl.BlockSpec((RB, D), lambda i: (i, 0)),
            pl.BlockSpec((RB, D), lambda i: (i, 0)),
            pl.BlockSpec((H, H), lambda i: (0, 0)),
            pl.BlockSpec((1, H), lambda i: (0, 0)),
            pl.BlockSpec((H, H), lambda i: (0, 0)),
        ],
        out_specs=pl.BlockSpec((RB, H), lambda i: (i, 0)),
        out_shape=jax.ShapeDtypeStruct((N, H), jnp.float32),
    )(agg2, cnt, h0, h1, W2l, b2.reshape(1, H), W2r)


def kernel(x, edge_index, W1l, b1, W1r, W2l, b2, W2r):
    ei = edge_index.astype(jnp.int32)
    src = ei[0]
    dst = ei[1]
    sum1, cnt = _sc_agg1(x, src, dst)
    h0, h1 = _tc_layer1(sum1, cnt, x, W1l, b1, W1r)
    agg2 = _sc_agg2(h0, h1, src, dst)
    return _tc_layer2(agg2, cnt, h0, h1, W2l, b2, W2r)


# speedup vs baseline: 7.6234x; 1.6391x over previous
import functools

import jax
import jax.numpy as jnp
from jax import lax
from jax.experimental import pallas as pl
from jax.experimental.pallas import tpu as pltpu
from jax.experimental.pallas import tpu_sc as plsc

N = 10000
E = 320000
D = 128
H = 256
NC = 2
NS = 16
CHUNK = 80
NPAD = 10240
ROWS_PER_SUB = NPAD // NS
ZCH = 128
RB = 1280


def _zero_acc_rows(zbuf, acc, s):
    @pl.loop(0, ZCH)
    def _(r):
        @pl.loop(0, D, step=16)
        def _(j):
            zbuf[r, pl.ds(j, 16)] = jnp.zeros((16,), jnp.float32)

    @pl.loop(0, ROWS_PER_SUB // ZCH)
    def _(j):
        pltpu.sync_copy(zbuf, acc.at[pl.ds(s * ROWS_PER_SUB + j * ZCH, ZCH)])


def _edge_pipeline(nch, base, src_hbm, dst_hbm, fire_gather, wait_gather,
                   sidx, didx, scatter):
    def load_and_fire(ci, b):
        e0 = base + ci * CHUNK
        pltpu.sync_copy(src_hbm.at[pl.ds(e0, CHUNK)], sidx[b])
        pltpu.sync_copy(dst_hbm.at[pl.ds(e0, CHUNK)], didx[b])
        fire_gather(b)

    load_and_fire(0, 0)

    @pl.loop(0, nch // 2)
    def _(j):
        c0 = 2 * j
        load_and_fire(c0 + 1, 1)
        wait_gather(0)
        scatter(0)

        @pl.when(c0 + 2 < nch)
        def _():
            load_and_fire(c0 + 2, 0)

        wait_gather(1)
        scatter(1)

    if nch % 2:
        wait_gather(0)
        scatter(0)


def _sc_agg1(x, src, dst):
    eps = E // (NC * NS)
    nch = eps // CHUNK
    mesh = plsc.VectorSubcoreMesh(core_axis_name="c", subcore_axis_name="s")

    @functools.partial(
        pl.kernel,
        out_type=[jax.ShapeDtypeStruct((NC, NPAD, D), jnp.float32),
                  jax.ShapeDtypeStruct((NC, NPAD), jnp.float32)],
        mesh=mesh,
        scratch_types=[
            pltpu.VMEM((CHUNK,), jnp.int32),
            pltpu.VMEM((CHUNK,), jnp.int32),
            pltpu.VMEM((CHUNK,), jnp.int32),
            pltpu.VMEM((CHUNK,), jnp.int32),
            pltpu.VMEM((CHUNK, D), jnp.float32),
            pltpu.VMEM((CHUNK, D), jnp.float32),
            pltpu.VMEM((CHUNK,), jnp.float32),
            pltpu.VMEM((ZCH, D), jnp.float32),
            pltpu.VMEM_SHARED((NPAD, D), jnp.float32),
            pltpu.VMEM_SHARED((NPAD,), jnp.float32),
            pltpu.SemaphoreType.DMA,
            pltpu.SemaphoreType.DMA,
        ],
    )
    def k(x_hbm, src_hbm, dst_hbm, osum_hbm, ocnt_hbm,
          sidx0, sidx1, didx0, didx1, rows0, rows1, ones, zbuf,
          acc, acc_cnt, sem0, sem1):
        c = lax.axis_index("c")
        s = lax.axis_index("s")
        sidx = (sidx0, sidx1)
        didx = (didx0, didx1)
        rows = (rows0, rows1)
        sem = (sem0, sem1)

        @pl.loop(0, CHUNK, step=16)
        def _(j):
            ones[pl.ds(j, 16)] = jnp.ones((16,), jnp.float32)

        _zero_acc_rows(zbuf, acc, s)

        @pl.loop(0, 5)
        def _(j):
            pltpu.sync_copy(zbuf.at[j, pl.ds(0, ZCH)],
                            acc_cnt.at[pl.ds(s * ROWS_PER_SUB + j * ZCH, ZCH)])

        plsc.subcore_barrier()

        base = (c * NS + s) * eps

        def fire(b):
            pltpu.async_copy(x_hbm.at[sidx[b]], rows[b], sem[b])

        def wait(b):
            pltpu.make_async_copy(x_hbm.at[sidx[b]], rows[b], sem[b]).wait()

        def scat(b):
            pltpu.sync_copy(rows[b], acc.at[didx[b]], add=True)
            pltpu.sync_copy(ones, acc_cnt.at[didx[b]], add=True)

        _edge_pipeline(nch, base, src_hbm, dst_hbm, fire, wait,
                       sidx, didx, scat)

        plsc.subcore_barrier()
        r0 = s * ROWS_PER_SUB
        pltpu.sync_copy(acc.at[pl.ds(r0, ROWS_PER_SUB)],
                        osum_hbm.at[c, pl.ds(r0, ROWS_PER_SUB)])
        pltpu.sync_copy(acc_cnt.at[pl.ds(r0, ROWS_PER_SUB)],
                        ocnt_hbm.at[c, pl.ds(r0, ROWS_PER_SUB)])

    return k(x, src, dst)


def _sc_agg2(h0, h1, src, dst):
    eps = E // NS
    nch = eps // CHUNK
    mesh = plsc.VectorSubcoreMesh(core_axis_name="c", subcore_axis_name="s")

    @functools.partial(
        pl.kernel,
        out_type=jax.ShapeDtypeStruct((NC, NPAD, D), jnp.float32),
        mesh=mesh,
        scratch_types=[
            pltpu.VMEM((CHUNK,), jnp.int32),
            pltpu.VMEM((CHUNK,), jnp.int32),
            pltpu.VMEM((CHUNK,), jnp.int32),
            pltpu.VMEM((CHUNK,), jnp.int32),
            pltpu.VMEM((CHUNK, D), jnp.float32),
            pltpu.VMEM((CHUNK, D), jnp.float32),
            pltpu.VMEM((ZCH, D), jnp.float32),
            pltpu.VMEM_SHARED((NPAD, D), jnp.float32),
            pltpu.SemaphoreType.DMA,
            pltpu.SemaphoreType.DMA,
        ],
    )
    def k(h0_hbm, h1_hbm, src_hbm, dst_hbm, out_hbm,
          sidx0, sidx1, didx0, didx1, rows0, rows1, zbuf, acc, sem0, sem1):
        c = lax.axis_index("c")
        s = lax.axis_index("s")
        sidx = (sidx0, sidx1)
        didx = (didx0, didx1)
        rows = (rows0, rows1)
        sem = (sem0, sem1)

        _zero_acc_rows(zbuf, acc, s)
        plsc.subcore_barrier()

        base = s * eps

        def scat(b):
            pltpu.sync_copy(rows[b], acc.at[didx[b]], add=True)

        @pl.when(c == 0)
        def _():
            def fire(b):
                pltpu.async_copy(h0_hbm.at[sidx[b]], rows[b], sem[b])

            def wait(b):
                pltpu.make_async_copy(h0_hbm.at[sidx[b]], rows[b],
                                      sem[b]).wait()

            _edge_pipeline(nch, base, src_hbm, dst_hbm, fire, wait,
                           sidx, didx, scat)

        @pl.when(c == 1)
        def _():
            def fire(b):
                pltpu.async_copy(h1_hbm.at[sidx[b]], rows[b], sem[b])

            def wait(b):
                pltpu.make_async_copy(h1_hbm.at[sidx[b]], rows[b],
                                      sem[b]).wait()

            _edge_pipeline(nch, base, src_hbm, dst_hbm, fire, wait,
                           sidx, didx, scat)

        plsc.subcore_barrier()
        r0 = s * ROWS_PER_SUB
        pltpu.sync_copy(acc.at[pl.ds(r0, ROWS_PER_SUB)],
                        out_hbm.at[c, pl.ds(r0, ROWS_PER_SUB)])

    return k(h0, h1, src, dst)


def _tc_layer1(sum1, cnt, x, W1l, b1, W1r):
    def body(a_ref, c_ref, x_ref, wl_ref, b_ref, wr_ref, h0_ref, h1_ref):
        ssum = a_ref[0] + a_ref[1]
        deg = c_ref[0] + c_ref[1]
        mean = ssum / jnp.clip(deg, 1.0)[:, None]
        h = jnp.dot(mean, wl_ref[...].T, preferred_element_type=jnp.float32)
        h = h + jnp.dot(x_ref[...], wr_ref[...].T,
                        preferred_element_type=jnp.float32)
        h = jnp.maximum(h + b_ref[...], 0.0)
        h0_ref[...] = h[:, :D]
        h1_ref[...] = h[:, D:]

    return pl.pallas_call(
        body,
        grid=(pl.cdiv(N, RB),),
        in_specs=[
            pl.BlockSpec((NC, RB, D), lambda i: (0, i, 0)),
            pl.BlockSpec((NC, RB), lambda i: (0, i)),
            pl.BlockSpec((RB, D), lambda i: (i, 0)),
            pl.BlockSpec((H, D), lambda i: (0, 0)),
            pl.BlockSpec((1, H), lambda i: (0, 0)),
            pl.BlockSpec((H, D), lambda i: (0, 0)),
        ],
        out_specs=[
            pl.BlockSpec((RB, D), lambda i: (i, 0)),
            pl.BlockSpec((RB, D), lambda i: (i, 0)),
        ],
        out_shape=[jax.ShapeDtypeStruct((N, D), jnp.float32),
                   jax.ShapeDtypeStruct((N, D), jnp.float32)],
    )(sum1, cnt, x, W1l, b1.reshape(1, H), W1r)


def _tc_layer2(agg2, cnt, h0, h1, W2l, b2, W2r):
    def body(g_ref, c_ref, h0_ref, h1_ref, wl_ref, b_ref, wr_ref, o_ref):
        deg = c_ref[0] + c_ref[1]
        inv = 1.0 / jnp.clip(deg, 1.0)
        m0 = g_ref[0] * inv[:, None]
        m1 = g_ref[1] * inv[:, None]
        wl = wl_ref[...]
        wr = wr_ref[...]
        o = jnp.dot(m0, wl[:, :D].T, preferred_element_type=jnp.float32)
        o = o + jnp.dot(m1, wl[:, D:].T, preferred_element_type=jnp.float32)
        o = o + jnp.dot(h0_ref[...], wr[:, :D].T,
                        preferred_element_type=jnp.float32)
        o = o + jnp.dot(h1_ref[...], wr[:, D:].T,
                        preferred_element_type=jnp.float32)
        o_ref[...] = o + b_ref[...]

    return pl.pallas_call(
        body,
        grid=(pl.cdiv(N, RB),),
        in_specs=[
            pl.BlockSpec((NC, RB, D), lambda i: (0, i, 0)),
            pl.BlockSpec((NC, RB), lambda i: (0, i)),
            pl.BlockSpec((RB, D), lambda i: (i, 0)),
            pl.BlockSpec((RB, D), lambda i: (i, 0)),
            pl.BlockSpec((H, H), lambda i: (0, 0)),
            pl.BlockSpec((1, H), lambda i: (0, 0)),
            pl.BlockSpec((H, H), lambda i: (0, 0)),
        ],
        out_specs=pl.BlockSpec((RB, H), lambda i: (i, 0)),
        out_shape=jax.ShapeDtypeStruct((N, H), jnp.float32),
    )(agg2, cnt, h0, h1, W2l, b2.reshape(1, H), W2r)


def kernel(x, edge_index, W1l, b1, W1r, W2l, b2, W2r):
    ei = edge_index.astype(jnp.int32)
    src = ei[0]
    dst = ei[1]
    sum1, cnt = _sc_agg1(x, src, dst)
    h0, h1 = _tc_layer1(sum1, cnt, x, W1l, b1, W1r)
    agg2 = _sc_agg2(h0, h1, src, dst)
    return _tc_layer2(agg2, cnt, h0, h1, W2l, b2, W2r)


# speedup vs baseline: 10.7989x; 1.4166x over previous
import functools

import jax
import jax.numpy as jnp
from jax import lax
from jax.experimental import pallas as pl
from jax.experimental.pallas import tpu as pltpu
from jax.experimental.pallas import tpu_sc as plsc

N = 10000
E = 320000
D = 128
H = 256
NC = 2
NS = 16
CHUNK = 80
NPAD = 10240
ROWS_PER_SUB = NPAD // NS
ZCH = 128
RB = 1280

NCH1 = E // (NC * NS) // CHUNK
NCH2 = E // NS // CHUNK
NBLK = 5
BLK1 = NCH1 // NBLK
BLK2 = NCH2 // NBLK


def _zero_acc_rows(zrows, acc, s):
    @pl.loop(0, CHUNK)
    def _(r):
        @pl.loop(0, D, step=16)
        def _(j):
            zrows[r, pl.ds(j, 16)] = jnp.zeros((16,), jnp.float32)

    @pl.loop(0, ROWS_PER_SUB // CHUNK)
    def _(j):
        pltpu.sync_copy(zrows,
                        acc.at[pl.ds(s * ROWS_PER_SUB + j * CHUNK, CHUNK)])


def _edge_pipeline(nch, fire_gather, wait_gather, scatter):
    fire_gather(0, 0)

    @pl.loop(0, nch // 2)
    def _(j):
        c0 = 2 * j
        fire_gather(c0 + 1, 1)
        wait_gather(c0, 0)
        scatter(c0, 0)

        @pl.when(c0 + 2 < nch)
        def _():
            fire_gather(c0 + 2, 0)

        wait_gather(c0 + 1, 1)
        scatter(c0 + 1, 1)

    if nch % 2:
        wait_gather(nch - 1, 0)
        scatter(nch - 1, 0)


def _sc_agg1(x, src3, dst3):
    mesh = plsc.VectorSubcoreMesh(core_axis_name="c", subcore_axis_name="s")

    @functools.partial(
        pl.kernel,
        out_type=[jax.ShapeDtypeStruct((NC, NPAD, D), jnp.float32),
                  jax.ShapeDtypeStruct((NC, NPAD), jnp.float32)],
        mesh=mesh,
        scratch_types=[
            pltpu.VMEM((BLK1, CHUNK), jnp.int32),
            pltpu.VMEM((BLK1, CHUNK), jnp.int32),
            pltpu.VMEM((CHUNK, D), jnp.float32),
            pltpu.VMEM((CHUNK, D), jnp.float32),
            pltpu.VMEM((CHUNK,), jnp.float32),
            pltpu.VMEM((ROWS_PER_SUB,), jnp.float32),
            pltpu.VMEM_SHARED((NPAD, D), jnp.float32),
            pltpu.VMEM_SHARED((NPAD,), jnp.float32),
            pltpu.SemaphoreType.DMA,
            pltpu.SemaphoreType.DMA,
        ],
    )
    def k(x_hbm, src_hbm, dst_hbm, osum_hbm, ocnt_hbm,
          sidx, didx, rows0, rows1, ones, zcnt, acc, acc_cnt, sem0, sem1):
        c = lax.axis_index("c")
        s = lax.axis_index("s")
        wid = c * NS + s
        rows = (rows0, rows1)
        sem = (sem0, sem1)

        @pl.loop(0, CHUNK, step=16)
        def _(j):
            ones[pl.ds(j, 16)] = jnp.ones((16,), jnp.float32)

        _zero_acc_rows(rows0, acc, s)

        @pl.loop(0, ROWS_PER_SUB, step=16)
        def _(j):
            zcnt[pl.ds(j, 16)] = jnp.zeros((16,), jnp.float32)

        pltpu.sync_copy(zcnt, acc_cnt.at[pl.ds(s * ROWS_PER_SUB,
                                               ROWS_PER_SUB)])

        plsc.subcore_barrier()

        def fire(ci, b):
            pltpu.async_copy(x_hbm.at[sidx.at[ci]], rows[b], sem[b])

        def wait(ci, b):
            pltpu.make_async_copy(x_hbm.at[sidx.at[ci]], rows[b],
                                  sem[b]).wait()

        def scat(ci, b):
            pltpu.sync_copy(rows[b], acc.at[didx.at[ci]], add=True)
            pltpu.sync_copy(ones, acc_cnt.at[didx.at[ci]], add=True)

        for blk in range(NBLK):
            pltpu.sync_copy(src_hbm.at[wid, blk], sidx)
            pltpu.sync_copy(dst_hbm.at[wid, blk], didx)
            _edge_pipeline(BLK1, fire, wait, scat)

        plsc.subcore_barrier()
        r0 = s * ROWS_PER_SUB
        pltpu.sync_copy(acc.at[pl.ds(r0, ROWS_PER_SUB)],
                        osum_hbm.at[c, pl.ds(r0, ROWS_PER_SUB)])
        pltpu.sync_copy(acc_cnt.at[pl.ds(r0, ROWS_PER_SUB)],
                        ocnt_hbm.at[c, pl.ds(r0, ROWS_PER_SUB)])

    return k(x, src3, dst3)


def _sc_agg2(h0, h1, src3, dst3):
    mesh = plsc.VectorSubcoreMesh(core_axis_name="c", subcore_axis_name="s")

    @functools.partial(
        pl.kernel,
        out_type=jax.ShapeDtypeStruct((NC, NPAD, D), jnp.float32),
        mesh=mesh,
        scratch_types=[
            pltpu.VMEM((BLK2, CHUNK), jnp.int32),
            pltpu.VMEM((BLK2, CHUNK), jnp.int32),
            pltpu.VMEM((CHUNK, D), jnp.float32),
            pltpu.VMEM((CHUNK, D), jnp.float32),
            pltpu.VMEM_SHARED((NPAD, D), jnp.float32),
            pltpu.SemaphoreType.DMA,
            pltpu.SemaphoreType.DMA,
        ],
    )
    def k(h0_hbm, h1_hbm, src_hbm, dst_hbm, out_hbm,
          sidx, didx, rows0, rows1, acc, sem0, sem1):
        c = lax.axis_index("c")
        s = lax.axis_index("s")
        rows = (rows0, rows1)
        sem = (sem0, sem1)

        _zero_acc_rows(rows0, acc, s)
        plsc.subcore_barrier()

        def scat(ci, b):
            pltpu.sync_copy(rows[b], acc.at[didx.at[ci]], add=True)

        for half in range(NBLK):
            pltpu.sync_copy(src_hbm.at[s, half], sidx)
            pltpu.sync_copy(dst_hbm.at[s, half], didx)

            @pl.when(c == 0)
            def _():
                def fire(ci, b):
                    pltpu.async_copy(h0_hbm.at[sidx.at[ci]], rows[b], sem[b])

                def wait(ci, b):
                    pltpu.make_async_copy(h0_hbm.at[sidx.at[ci]], rows[b],
                                          sem[b]).wait()

                _edge_pipeline(BLK2, fire, wait, scat)

            @pl.when(c == 1)
            def _():
                def fire(ci, b):
                    pltpu.async_copy(h1_hbm.at[sidx.at[ci]], rows[b], sem[b])

                def wait(ci, b):
                    pltpu.make_async_copy(h1_hbm.at[sidx.at[ci]], rows[b],
                                          sem[b]).wait()

                _edge_pipeline(BLK2, fire, wait, scat)

        plsc.subcore_barrier()
        r0 = s * ROWS_PER_SUB
        pltpu.sync_copy(acc.at[pl.ds(r0, ROWS_PER_SUB)],
                        out_hbm.at[c, pl.ds(r0, ROWS_PER_SUB)])

    return k(h0, h1, src3, dst3)


def _tc_layer1(sum1, cnt, x, W1l, b1, W1r):
    def body(a_ref, c_ref, x_ref, wl_ref, b_ref, wr_ref, h0_ref, h1_ref):
        ssum = a_ref[0] + a_ref[1]
        deg = c_ref[0] + c_ref[1]
        mean = ssum / jnp.clip(deg, 1.0)[:, None]
        h = jnp.dot(mean, wl_ref[...].T, preferred_element_type=jnp.float32)
        h = h + jnp.dot(x_ref[...], wr_ref[...].T,
                        preferred_element_type=jnp.float32)
        h = jnp.maximum(h + b_ref[...], 0.0)
        h0_ref[...] = h[:, :D]
        h1_ref[...] = h[:, D:]

    return pl.pallas_call(
        body,
        grid=(pl.cdiv(N, RB),),
        in_specs=[
            pl.BlockSpec((NC, RB, D), lambda i: (0, i, 0)),
            pl.BlockSpec((NC, RB), lambda i: (0, i)),
            pl.BlockSpec((RB, D), lambda i: (i, 0)),
            pl.BlockSpec((H, D), lambda i: (0, 0)),
            pl.BlockSpec((1, H), lambda i: (0, 0)),
            pl.BlockSpec((H, D), lambda i: (0, 0)),
        ],
        out_specs=[
            pl.BlockSpec((RB, D), lambda i: (i, 0)),
            pl.BlockSpec((RB, D), lambda i: (i, 0)),
        ],
        out_shape=[jax.ShapeDtypeStruct((N, D), jnp.float32),
                   jax.ShapeDtypeStruct((N, D), jnp.float32)],
    )(sum1, cnt, x, W1l, b1.reshape(1, H), W1r)


def _tc_layer2(agg2, cnt, h0, h1, W2l, b2, W2r):
    def body(g_ref, c_ref, h0_ref, h1_ref, wl_ref, b_ref, wr_ref, o_ref):
        deg = c_ref[0] + c_ref[1]
        inv = 1.0 / jnp.clip(deg, 1.0)
        m0 = g_ref[0] * inv[:, None]
        m1 = g_ref[1] * inv[:, None]
        wl = wl_ref[...]
        wr = wr_ref[...]
        o = jnp.dot(m0, wl[:, :D].T, preferred_element_type=jnp.float32)
        o = o + jnp.dot(m1, wl[:, D:].T, preferred_element_type=jnp.float32)
        o = o + jnp.dot(h0_ref[...], wr[:, :D].T,
                        preferred_element_type=jnp.float32)
        o = o + jnp.dot(h1_ref[...], wr[:, D:].T,
                        preferred_element_type=jnp.float32)
        o_ref[...] = o + b_ref[...]

    return pl.pallas_call(
        body,
        grid=(pl.cdiv(N, RB),),
        in_specs=[
            pl.BlockSpec((NC, RB, D), lambda i: (0, i, 0)),
            pl.BlockSpec((NC, RB), lambda i: (0, i)),
            pl.BlockSpec((RB, D), lambda i: (i, 0)),
            pl.BlockSpec((RB, D), lambda i: (i, 0)),
            pl.BlockSpec((H, H), lambda i: (0, 0)),
            pl.BlockSpec((1, H), lambda i: (0, 0)),
            pl.BlockSpec((H, H), lambda i: (0, 0)),
        ],
        out_specs=pl.BlockSpec((RB, H), lambda i: (i, 0)),
        out_shape=jax.ShapeDtypeStruct((N, H), jnp.float32),
    )(agg2, cnt, h0, h1, W2l, b2.reshape(1, H), W2r)


def kernel(x, edge_index, W1l, b1, W1r, W2l, b2, W2r):
    ei = edge_index.astype(jnp.int32)
    src = ei[0]
    dst = ei[1]
    src3a = src.reshape(NC * NS, NBLK, BLK1, CHUNK)
    dst3a = dst.reshape(NC * NS, NBLK, BLK1, CHUNK)
    src3b = src.reshape(NS, NBLK, BLK2, CHUNK)
    dst3b = dst.reshape(NS, NBLK, BLK2, CHUNK)
    sum1, cnt = _sc_agg1(x, src3a, dst3a)
    h0, h1 = _tc_layer1(sum1, cnt, x, W1l, b1, W1r)
    agg2 = _sc_agg2(h0, h1, src3b, dst3b)
    return _tc_layer2(agg2, cnt, h0, h1, W2l, b2, W2r)


# speedup vs baseline: 12.1812x; 1.1280x over previous
import functools

import jax
import jax.numpy as jnp
from jax import lax
from jax.experimental import pallas as pl
from jax.experimental.pallas import tpu as pltpu
from jax.experimental.pallas import tpu_sc as plsc

N = 10000
E = 320000
D = 128
H = 256
NC = 2
NS = 16
CHUNK = 80
NPAD = 10240
ROWS_PER_SUB = NPAD // NS
ZCH = 128
RB = 1280

NCH1 = E // (NC * NS) // CHUNK
NCH2 = E // NS // CHUNK
NBLK1 = 5
NBLK2 = 10
BLK1 = NCH1 // NBLK1
BLK2 = NCH2 // NBLK2


def _zero_acc_rows(zrows, acc, s):
    @pl.loop(0, CHUNK)
    def _(r):
        @pl.loop(0, D, step=16)
        def _(j):
            zrows[r, pl.ds(j, 16)] = jnp.zeros((16,), jnp.float32)

    @pl.loop(0, ROWS_PER_SUB // CHUNK)
    def _(j):
        pltpu.sync_copy(zrows,
                        acc.at[pl.ds(s * ROWS_PER_SUB + j * CHUNK, CHUNK)])


NBUF = 3


def _edge_pipeline(nch, fire_gather, wait_gather, scatter):
    for b in range(NBUF):
        fire_gather(b, b)

    @pl.loop(0, nch // NBUF)
    def _(j):
        c0 = NBUF * j
        for b in range(NBUF):
            wait_gather(c0 + b, b)
            scatter(c0 + b, b)

            @pl.when(c0 + b + NBUF < nch)
            def _():
                fire_gather(c0 + b + NBUF, b)

    tail = nch % NBUF
    for r in range(tail):
        wait_gather(nch - tail + r, r)
        scatter(nch - tail + r, r)


def _sc_agg1(x, src3, dst3):
    mesh = plsc.VectorSubcoreMesh(core_axis_name="c", subcore_axis_name="s")

    @functools.partial(
        pl.kernel,
        out_type=[jax.ShapeDtypeStruct((NC, NPAD, D), jnp.float32),
                  jax.ShapeDtypeStruct((NC, NPAD), jnp.float32)],
        mesh=mesh,
        scratch_types=[
            pltpu.VMEM((BLK1, CHUNK), jnp.int32),
            pltpu.VMEM((BLK1, CHUNK), jnp.int32),
            pltpu.VMEM((CHUNK, D), jnp.float32),
            pltpu.VMEM((CHUNK, D), jnp.float32),
            pltpu.VMEM((CHUNK, D), jnp.float32),
            pltpu.VMEM((CHUNK,), jnp.float32),
            pltpu.VMEM((ROWS_PER_SUB,), jnp.float32),
            pltpu.VMEM_SHARED((NPAD, D), jnp.float32),
            pltpu.VMEM_SHARED((NPAD,), jnp.float32),
            pltpu.SemaphoreType.DMA,
            pltpu.SemaphoreType.DMA,
            pltpu.SemaphoreType.DMA,
        ],
    )
    def k(x_hbm, src_hbm, dst_hbm, osum_hbm, ocnt_hbm,
          sidx, didx, rows0, rows1, rows2, ones, zcnt, acc, acc_cnt,
          sem0, sem1, sem2):
        c = lax.axis_index("c")
        s = lax.axis_index("s")
        wid = c * NS + s
        rows = (rows0, rows1, rows2)
        sem = (sem0, sem1, sem2)

        @pl.loop(0, CHUNK, step=16)
        def _(j):
            ones[pl.ds(j, 16)] = jnp.ones((16,), jnp.float32)

        _zero_acc_rows(rows0, acc, s)

        @pl.loop(0, ROWS_PER_SUB, step=16)
        def _(j):
            zcnt[pl.ds(j, 16)] = jnp.zeros((16,), jnp.float32)

        pltpu.sync_copy(zcnt, acc_cnt.at[pl.ds(s * ROWS_PER_SUB,
                                               ROWS_PER_SUB)])

        plsc.subcore_barrier()

        def fire(ci, b):
            pltpu.async_copy(x_hbm.at[sidx.at[ci]], rows[b], sem[b])

        def wait(ci, b):
            pltpu.make_async_copy(x_hbm.at[sidx.at[ci]], rows[b],
                                  sem[b]).wait()

        def scat(ci, b):
            pltpu.sync_copy(rows[b], acc.at[didx.at[ci]], add=True)
            pltpu.sync_copy(ones, acc_cnt.at[didx.at[ci]], add=True)

        for blk in range(NBLK1):
            pltpu.sync_copy(src_hbm.at[wid, blk], sidx)
            pltpu.sync_copy(dst_hbm.at[wid, blk], didx)
            _edge_pipeline(BLK1, fire, wait, scat)

        plsc.subcore_barrier()
        r0 = s * ROWS_PER_SUB
        pltpu.sync_copy(acc.at[pl.ds(r0, ROWS_PER_SUB)],
                        osum_hbm.at[c, pl.ds(r0, ROWS_PER_SUB)])
        pltpu.sync_copy(acc_cnt.at[pl.ds(r0, ROWS_PER_SUB)],
                        ocnt_hbm.at[c, pl.ds(r0, ROWS_PER_SUB)])

    return k(x, src3, dst3)


def _sc_agg2(h0, h1, src3, dst3):
    mesh = plsc.VectorSubcoreMesh(core_axis_name="c", subcore_axis_name="s")

    @functools.partial(
        pl.kernel,
        out_type=jax.ShapeDtypeStruct((NC, NPAD, D), jnp.float32),
        mesh=mesh,
        scratch_types=[
            pltpu.VMEM((BLK2, CHUNK), jnp.int32),
            pltpu.VMEM((BLK2, CHUNK), jnp.int32),
            pltpu.VMEM((CHUNK, D), jnp.float32),
            pltpu.VMEM((CHUNK, D), jnp.float32),
            pltpu.VMEM((CHUNK, D), jnp.float32),
            pltpu.VMEM_SHARED((NPAD, D), jnp.float32),
            pltpu.SemaphoreType.DMA,
            pltpu.SemaphoreType.DMA,
            pltpu.SemaphoreType.DMA,
        ],
    )
    def k(h0_hbm, h1_hbm, src_hbm, dst_hbm, out_hbm,
          sidx, didx, rows0, rows1, rows2, acc, sem0, sem1, sem2):
        c = lax.axis_index("c")
        s = lax.axis_index("s")
        rows = (rows0, rows1, rows2)
        sem = (sem0, sem1, sem2)

        _zero_acc_rows(rows0, acc, s)
        plsc.subcore_barrier()

        def scat(ci, b):
            pltpu.sync_copy(rows[b], acc.at[didx.at[ci]], add=True)

        for half in range(NBLK2):
            pltpu.sync_copy(src_hbm.at[s, half], sidx)
            pltpu.sync_copy(dst_hbm.at[s, half], didx)

            @pl.when(c == 0)
            def _():
                def fire(ci, b):
                    pltpu.async_copy(h0_hbm.at[sidx.at[ci]], rows[b], sem[b])

                def wait(ci, b):
                    pltpu.make_async_copy(h0_hbm.at[sidx.at[ci]], rows[b],
                                          sem[b]).wait()

                _edge_pipeline(BLK2, fire, wait, scat)

            @pl.when(c == 1)
            def _():
                def fire(ci, b):
                    pltpu.async_copy(h1_hbm.at[sidx.at[ci]], rows[b], sem[b])

                def wait(ci, b):
                    pltpu.make_async_copy(h1_hbm.at[sidx.at[ci]], rows[b],
                                          sem[b]).wait()

                _edge_pipeline(BLK2, fire, wait, scat)

        plsc.subcore_barrier()
        r0 = s * ROWS_PER_SUB
        pltpu.sync_copy(acc.at[pl.ds(r0, ROWS_PER_SUB)],
                        out_hbm.at[c, pl.ds(r0, ROWS_PER_SUB)])

    return k(h0, h1, src3, dst3)


def _tc_layer1(sum1, cnt, x, W1l, b1, W1r):
    def body(a_ref, c_ref, x_ref, wl_ref, b_ref, wr_ref, h0_ref, h1_ref):
        ssum = a_ref[0] + a_ref[1]
        deg = c_ref[0] + c_ref[1]
        mean = ssum / jnp.clip(deg, 1.0)[:, None]
        h = jnp.dot(mean, wl_ref[...].T, preferred_element_type=jnp.float32)
        h = h + jnp.dot(x_ref[...], wr_ref[...].T,
                        preferred_element_type=jnp.float32)
        h = jnp.maximum(h + b_ref[...], 0.0)
        h0_ref[...] = h[:, :D]
        h1_ref[...] = h[:, D:]

    return pl.pallas_call(
        body,
        grid=(pl.cdiv(N, RB),),
        in_specs=[
            pl.BlockSpec((NC, RB, D), lambda i: (0, i, 0)),
            pl.BlockSpec((NC, RB), lambda i: (0, i)),
            pl.BlockSpec((RB, D), lambda i: (i, 0)),
            pl.BlockSpec((H, D), lambda i: (0, 0)),
            pl.BlockSpec((1, H), lambda i: (0, 0)),
            pl.BlockSpec((H, D), lambda i: (0, 0)),
        ],
        out_specs=[
            pl.BlockSpec((RB, D), lambda i: (i, 0)),
            pl.BlockSpec((RB, D), lambda i: (i, 0)),
        ],
        out_shape=[jax.ShapeDtypeStruct((N, D), jnp.float32),
                   jax.ShapeDtypeStruct((N, D), jnp.float32)],
    )(sum1, cnt, x, W1l, b1.reshape(1, H), W1r)


def _tc_layer2(agg2, cnt, h0, h1, W2l, b2, W2r):
    def body(g_ref, c_ref, h0_ref, h1_ref, wl_ref, b_ref, wr_ref, o_ref):
        deg = c_ref[0] + c_ref[1]
        inv = 1.0 / jnp.clip(deg, 1.0)
        m0 = g_ref[0] * inv[:, None]
        m1 = g_ref[1] * inv[:, None]
        wl = wl_ref[...]
        wr = wr_ref[...]
        o = jnp.dot(m0, wl[:, :D].T, preferred_element_type=jnp.float32)
        o = o + jnp.dot(m1, wl[:, D:].T, preferred_element_type=jnp.float32)
        o = o + jnp.dot(h0_ref[...], wr[:, :D].T,
                        preferred_element_type=jnp.float32)
        o = o + jnp.dot(h1_ref[...], wr[:, D:].T,
                        preferred_element_type=jnp.float32)
        o_ref[...] = o + b_ref[...]

    return pl.pallas_call(
        body,
        grid=(pl.cdiv(N, RB),),
        in_specs=[
            pl.BlockSpec((NC, RB, D), lambda i: (0, i, 0)),
            pl.BlockSpec((NC, RB), lambda i: (0, i)),
            pl.BlockSpec((RB, D), lambda i: (i, 0)),
            pl.BlockSpec((RB, D), lambda i: (i, 0)),
            pl.BlockSpec((H, H), lambda i: (0, 0)),
            pl.BlockSpec((1, H), lambda i: (0, 0)),
            pl.BlockSpec((H, H), lambda i: (0, 0)),
        ],
        out_specs=pl.BlockSpec((RB, H), lambda i: (i, 0)),
        out_shape=jax.ShapeDtypeStruct((N, H), jnp.float32),
    )(agg2, cnt, h0, h1, W2l, b2.reshape(1, H), W2r)


def kernel(x, edge_index, W1l, b1, W1r, W2l, b2, W2r):
    ei = edge_index.astype(jnp.int32)
    src = ei[0]
    dst = ei[1]
    src3a = src.reshape(NC * NS, NBLK1, BLK1, CHUNK)
    dst3a = dst.reshape(NC * NS, NBLK1, BLK1, CHUNK)
    src3b = src.reshape(NS, NBLK2, BLK2, CHUNK)
    dst3b = dst.reshape(NS, NBLK2, BLK2, CHUNK)
    sum1, cnt = _sc_agg1(x, src3a, dst3a)
    h0, h1 = _tc_layer1(sum1, cnt, x, W1l, b1, W1r)
    agg2 = _sc_agg2(h0, h1, src3b, dst3b)
    return _tc_layer2(agg2, cnt, h0, h1, W2l, b2, W2r)
